# Initial kernel scaffold; baseline (speedup 1.0000x reference)
import jax
import jax.numpy as jnp
from jax.experimental import pallas as pl


def kernel(edge_index, edge_values, embeds):
    raise NotImplementedError("write your pallas kernel here")


# speedup vs baseline: 4.4756x; 4.4756x over previous
import functools

import jax
import jax.numpy as jnp
from jax import lax
from jax.experimental import pallas as pl
from jax.experimental.pallas import tpu as pltpu
from jax.experimental.pallas import tpu_sc as plsc

N_NODES = 10000
D_FEAT = 128
NUM_CORES = 2
NUM_SUBCORES = 16
NUM_WORKERS = NUM_CORES * NUM_SUBCORES
CHUNK = 80
ROWS_PER_SUB = N_NODES // NUM_SUBCORES


def _sc_spmm(src, dst, vals, embeds, zeros):
    num_edges = src.shape[0]
    edges_per_worker = num_edges // NUM_WORKERS
    num_chunks = edges_per_worker // CHUNK

    mesh = plsc.VectorSubcoreMesh(core_axis_name="c", subcore_axis_name="s")

    @functools.partial(
        pl.kernel,
        mesh=mesh,
        out_type=jax.ShapeDtypeStruct((NUM_CORES, N_NODES, D_FEAT), jnp.float32),
        scratch_types=[
            pltpu.VMEM((CHUNK,), jnp.int32),
            pltpu.VMEM((CHUNK,), jnp.int32),
            pltpu.VMEM((CHUNK,), jnp.float32),
            pltpu.VMEM((CHUNK, D_FEAT), jnp.float32),
            pltpu.VMEM_SHARED((N_NODES, D_FEAT), jnp.float32),
            pltpu.SemaphoreType.DMA,
        ],
    )
    def k(src_hbm, dst_hbm, val_hbm, emb_hbm, zero_hbm, out_hbm,
          src_v, dst_v, val_v, rows_v, acc_sh, sem):
        cid = lax.axis_index("c")
        sid = lax.axis_index("s")
        wid = cid * NUM_SUBCORES + sid

        row0 = sid * 624

        @pl.when(sid < NUM_SUBCORES - 1)
        def _():
            pltpu.sync_copy(zero_hbm.at[pl.ds(row0, 624)],
                            acc_sh.at[pl.ds(row0, 624)])

        @pl.when(sid == NUM_SUBCORES - 1)
        def _():
            pltpu.sync_copy(zero_hbm.at[pl.ds(15 * 624, 640)],
                            acc_sh.at[pl.ds(15 * 624, 640)])

        plsc.subcore_barrier()

        base_w = wid * edges_per_worker

        @pl.loop(0, num_chunks)
        def _(i):
            base = base_w + i * CHUNK
            pltpu.sync_copy(src_hbm.at[pl.ds(base, CHUNK)], src_v)
            pltpu.sync_copy(dst_hbm.at[pl.ds(base, CHUNK)], dst_v)
            pltpu.sync_copy(val_hbm.at[pl.ds(base, CHUNK)], val_v)
            pltpu.async_copy(emb_hbm.at[src_v], rows_v, sem).wait()

            @pl.loop(0, CHUNK // 16)
            def _(g):
                v16 = val_v[pl.ds(g * 16, 16)]
                for j in range(16):
                    v = v16[j]
                    e = g * 16 + j
                    for f in range(D_FEAT // 16):
                        sl = pl.ds(f * 16, 16)
                        rows_v[e, sl] = rows_v[e, sl] * v

            pltpu.sync_copy(rows_v, acc_sh.at[dst_v], add=True)

        plsc.subcore_barrier()

        @pl.when(sid < NUM_SUBCORES - 1)
        def _():
            pltpu.sync_copy(acc_sh.at[pl.ds(row0, 624)],
                            out_hbm.at[cid, pl.ds(row0, 624)])

        @pl.when(sid == NUM_SUBCORES - 1)
        def _():
            pltpu.sync_copy(acc_sh.at[pl.ds(15 * 624, 640)],
                            out_hbm.at[cid, pl.ds(15 * 624, 640)])

    return k(src, dst, vals, embeds, zeros)


def _tc_combine(partials):
    def body(a_ref, b_ref, o_ref):
        o_ref[...] = a_ref[0] + b_ref[0]

    blk = 1000
    return pl.pallas_call(
        body,
        out_shape=jax.ShapeDtypeStruct((N_NODES, D_FEAT), jnp.float32),
        grid=(N_NODES // blk,),
        in_specs=[
            pl.BlockSpec((1, blk, D_FEAT), lambda i: (0, i, 0)),
            pl.BlockSpec((1, blk, D_FEAT), lambda i: (1, i, 0)),
        ],
        out_specs=pl.BlockSpec((blk, D_FEAT), lambda i: (i, 0)),
    )(partials, partials)


@jax.jit
def kernel(edge_index, edge_values, embeds):
    dst = edge_index[0].astype(jnp.int32)
    src = edge_index[1].astype(jnp.int32)
    vals = edge_values.astype(jnp.float32)
    zeros = jnp.zeros((N_NODES, D_FEAT), jnp.float32)
    partials = _sc_spmm(src, dst, vals, embeds, zeros)
    return _tc_combine(partials)
